# baseline (speedup 1.0000x reference)
import jax
import jax.numpy as jnp
import numpy as np
from jax import lax
from jax.experimental import pallas as pl
from jax.experimental.pallas import tpu as pltpu
from jax.experimental.pallas import tpu_sc as plsc

B = 8
B_H = 4
N = 1024
K = 16
C_IN = 64
C_OUT = 128
C_LIFT = 64
TILE = 512
Q = N // 4
BIG = 1e9
HUGE = 3e9
IDX_BIG = 1e9
HI = jax.lax.Precision.HIGHEST
PAD_D = 128
ROWS = B_H * N
LOOKUPS = B_H * N * K
NW = 32
PER_W = LOOKUPS // NW
CHUNK = 256
NCHUNK = PER_W // CHUNK


def _ce(va, ia, vb, ib):
    pred = (va > vb) | ((va == vb) & (ia > ib))
    return (jnp.where(pred, vb, va), jnp.where(pred, ib, ia),
            jnp.where(pred, va, vb), jnp.where(pred, ia, ib))


def _topk_kernel(q_ref, p_ref, pT_ref, feat_ref, Wl_ref, bl_ref,
                 idx_ref, payload_ref):
    b = pl.program_id(0)
    q = q_ref[0]
    p_full = p_ref[0]
    pT = pT_ref[0]
    feat = feat_ref[0]

    p0 = pT[0:1, :]
    p1 = pT[1:2, :]
    p2 = pT[2:3, :]
    valid_p = (p0 != 0.0) | (p1 != 0.0) | (p2 != 0.0)
    valid_col = jnp.any(p_full != 0.0, axis=1, keepdims=True)
    valid_q = jnp.any(q != 0.0, axis=1, keepdims=True)

    d0 = q[:, 0:1] - p0
    d1 = q[:, 1:2] - p1
    d2 = q[:, 2:3] - p2
    pd = d0 * d0 + d1 * d1 + d2 * d2
    pd = jnp.where(valid_q & valid_p, pd, BIG)

    @pl.when(pl.program_id(1) == 0)
    def _build_payload():
        lifted = jax.nn.relu(jnp.dot(feat, Wl_ref[...],
                                     preferred_element_type=jnp.float32)
                             + bl_ref[...])
        lifted = jnp.where(valid_col, lifted, 0.0)
        payload_ref[0] = jnp.concatenate(
            [p_full, valid_col.astype(jnp.float32), lifted,
             jnp.zeros((N, PAD_D - 4 - C_LIFT), jnp.float32)], axis=1)

    base = jax.lax.broadcasted_iota(jnp.int32, (TILE, Q), 1).astype(jnp.float32)
    v0, i0 = pd[:, 0:Q], base
    v1, i1 = pd[:, Q:2 * Q], base + float(Q)
    v2, i2 = pd[:, 2 * Q:3 * Q], base + float(2 * Q)
    v3, i3 = pd[:, 3 * Q:4 * Q], base + float(3 * Q)
    v0, i0, v1, i1 = _ce(v0, i0, v1, i1)
    v2, i2, v3, i3 = _ce(v2, i2, v3, i3)
    v0, i0, v2, i2 = _ce(v0, i0, v2, i2)
    v1, i1, v3, i3 = _ce(v1, i1, v3, i3)
    v1, i1, v2, i2 = _ce(v1, i1, v2, i2)

    cols = []
    for _ in range(K):
        m = jnp.min(v0, axis=1, keepdims=True)
        cand = jnp.where(v0 <= m, i0, IDX_BIG)
        argi = jnp.min(cand, axis=1, keepdims=True)
        pool_oh = i0 == argi
        v0 = jnp.where(pool_oh, v1, v0)
        i0 = jnp.where(pool_oh, i1, i0)
        v1 = jnp.where(pool_oh, v2, v1)
        i1 = jnp.where(pool_oh, i2, i1)
        v2 = jnp.where(pool_oh, v3, v2)
        i2 = jnp.where(pool_oh, i3, i2)
        v3 = jnp.where(pool_oh, HUGE, v3)
        i3 = jnp.where(pool_oh, IDX_BIG, i3)
        cols.append(argi)
    idx_f = jnp.concatenate(cols, axis=1)
    idx_ref[0] = idx_f.astype(jnp.int32) + b * N


def _sc_gather(table_ref, idx_ref, out_ref, idx_v0, idx_v1, rows_v0, rows_v1,
               semg0, semg1, semo0, semo1):
    wid = lax.axis_index("s") * 2 + lax.axis_index("c")
    base = wid * PER_W
    idx_bufs = (idx_v0, idx_v1)
    row_bufs = (rows_v0, rows_v1)
    gsems = (semg0, semg1)
    osems = (semo0, semo1)
    gathers = [None] * NCHUNK
    outs = [None] * NCHUNK
    for ci in range(NCHUNK):
        bf = ci % 2
        start = base + ci * CHUNK
        if ci >= 2:
            outs[ci - 2].wait()
        pltpu.sync_copy(idx_ref.at[pl.ds(start, CHUNK)], idx_bufs[bf])
        gathers[ci] = pltpu.async_copy(table_ref.at[idx_bufs[bf]],
                                       row_bufs[bf], gsems[bf])
        if ci >= 1:
            gathers[ci - 1].wait()
            pstart = base + (ci - 1) * CHUNK
            outs[ci - 1] = pltpu.async_copy(
                row_bufs[1 - bf], out_ref.at[pl.ds(pstart, CHUNK)],
                osems[1 - bf])
    gathers[NCHUNK - 1].wait()
    last = base + (NCHUNK - 1) * CHUNK
    outs[NCHUNK - 1] = pltpu.async_copy(
        row_bufs[(NCHUNK - 1) % 2], out_ref.at[pl.ds(last, CHUNK)],
        osems[(NCHUNK - 1) % 2])
    outs[NCHUNK - 2].wait()
    outs[NCHUNK - 1].wait()


def _dense_kernel(q_ref, g_ref, W1_ref, b1_ref, W2_ref, b2_ref,
                  Wf_ref, bf_ref, out_ref):
    q = q_ref[0]
    gb = g_ref[0]
    valid_q = jnp.any(q != 0.0, axis=1, keepdims=True)

    h_parts = []
    L_parts = []
    for k in range(K):
        g = gb[k]
        nbr_valid = g[:, 3:4] > 0.5
        rel = jnp.where(nbr_valid, g[:, 0:3] - q, 0.0)
        h = jax.nn.relu(jnp.dot(rel, W1_ref[...],
                                preferred_element_type=jnp.float32)
                        + b1_ref[...])
        h_parts.append(h)
        L_parts.append(g[:, 4:4 + C_LIFT])

    h_flat = jnp.concatenate(h_parts, axis=1)
    Xp = jnp.dot(h_flat, W2_ref[...],
                 preferred_element_type=jnp.float32) + b2_ref[...]

    lane_e = jax.lax.broadcasted_iota(jnp.int32, (K, N), 1)
    row_e = jax.lax.broadcasted_iota(jnp.int32, (K, N), 0)
    E = (lane_e // C_LIFT == row_e).astype(jnp.float32)

    T = None
    for j in range(K):
        Xrep = jnp.dot(Xp[:, j * K:(j + 1) * K], E,
                       preferred_element_type=jnp.float32, precision=HI)
        Ltile = pltpu.repeat(L_parts[j], K, axis=1)
        T = Xrep * Ltile if T is None else T + Xrep * Ltile

    final = jax.nn.relu(jnp.dot(T, Wf_ref[...],
                                preferred_element_type=jnp.float32)
                        + bf_ref[...])
    out_ref[0] = jnp.where(valid_q, final, 0.0)


def _half_pipeline(points_xyz, features, W1, b1, W2p, b2p, Wl, bl, Wf, bf):
    pT = jnp.transpose(points_xyz, (0, 2, 1))

    idx, payload = pl.pallas_call(
        _topk_kernel,
        grid=(B_H, N // TILE),
        in_specs=[
            pl.BlockSpec((1, TILE, 3), lambda b, t: (b, t, 0)),
            pl.BlockSpec((1, N, 3), lambda b, t: (b, 0, 0)),
            pl.BlockSpec((1, 3, N), lambda b, t: (b, 0, 0)),
            pl.BlockSpec((1, N, C_IN), lambda b, t: (b, 0, 0)),
            pl.BlockSpec(Wl.shape, lambda b, t: (0, 0)),
            pl.BlockSpec((1, C_LIFT), lambda b, t: (0, 0)),
        ],
        out_specs=[
            pl.BlockSpec((1, TILE, K), lambda b, t: (b, t, 0)),
            pl.BlockSpec((1, N, PAD_D), lambda b, t: (b, 0, 0)),
        ],
        out_shape=[
            jax.ShapeDtypeStruct((B_H, N, K), jnp.int32),
            jax.ShapeDtypeStruct((B_H, N, PAD_D), jnp.float32),
        ],
    )(points_xyz, points_xyz, pT, features, Wl, bl.reshape(1, -1))

    table = payload.reshape(ROWS, PAD_D)
    idx_flat = jnp.transpose(idx, (0, 2, 1)).reshape(LOOKUPS)

    mesh = plsc.VectorSubcoreMesh(core_axis_name="c", subcore_axis_name="s")
    gathered = pl.kernel(
        _sc_gather,
        mesh=mesh,
        out_type=jax.ShapeDtypeStruct((LOOKUPS, PAD_D), jnp.float32),
        scratch_types=[
            pltpu.VMEM((CHUNK,), jnp.int32),
            pltpu.VMEM((CHUNK,), jnp.int32),
            pltpu.VMEM((CHUNK, PAD_D), jnp.float32),
            pltpu.VMEM((CHUNK, PAD_D), jnp.float32),
            pltpu.SemaphoreType.DMA,
            pltpu.SemaphoreType.DMA,
            pltpu.SemaphoreType.DMA,
            pltpu.SemaphoreType.DMA,
        ],
    )(table, idx_flat)

    g4 = gathered.reshape(B_H, K, N, PAD_D)

    out = pl.pallas_call(
        _dense_kernel,
        grid=(B_H, N // TILE),
        in_specs=[
            pl.BlockSpec((1, TILE, 3), lambda b, t: (b, t, 0)),
            pl.BlockSpec((1, K, TILE, PAD_D), lambda b, t: (b, 0, t, 0)),
            pl.BlockSpec(W1.shape, lambda b, t: (0, 0)),
            pl.BlockSpec((1, K * 2), lambda b, t: (0, 0)),
            pl.BlockSpec(W2p.shape, lambda b, t: (0, 0)),
            pl.BlockSpec((1, K * K), lambda b, t: (0, 0)),
            pl.BlockSpec(Wf.shape, lambda b, t: (0, 0)),
            pl.BlockSpec((1, C_OUT), lambda b, t: (0, 0)),
        ],
        out_specs=pl.BlockSpec((1, TILE, C_OUT), lambda b, t: (b, t, 0)),
        out_shape=jax.ShapeDtypeStruct((B_H, N, C_OUT), jnp.float32),
    )(points_xyz, g4, W1, b1.reshape(1, -1), W2p, b2p.reshape(1, -1),
      Wf, bf.reshape(1, -1))
    return out


@jax.jit
def kernel(points_xyz, features, W1, b1, W2, b2, Wl, bl, Wf, bf):
    perm = np.arange(K * K).reshape(K, K).T.reshape(-1)
    W2p = W2[:, perm]
    b2p = b2[perm]
    halves = [
        _half_pipeline(points_xyz[h * B_H:(h + 1) * B_H],
                       features[h * B_H:(h + 1) * B_H],
                       W1, b1, W2p, b2p, Wl, bl, Wf, bf)
        for h in range(B // B_H)
    ]
    return jnp.concatenate(halves, axis=0)


# speedup vs baseline: 1.0042x; 1.0042x over previous
import jax
import jax.numpy as jnp
import numpy as np
from jax import lax
from jax.experimental import pallas as pl
from jax.experimental.pallas import tpu as pltpu
from jax.experimental.pallas import tpu_sc as plsc

B = 8
B_H = 4
N = 1024
K = 16
C_IN = 64
C_OUT = 128
C_LIFT = 64
TILE = 512
Q = N // 4
BIG = 1e9
HUGE = 3e9
IDX_BIG = 1e9
HI = jax.lax.Precision.HIGHEST
PAD_D = 128
ROWS = B_H * N
LOOKUPS = B_H * N * K
NW = 32
PER_W = LOOKUPS // NW
CHUNK = 512
NCHUNK = PER_W // CHUNK


def _ce(va, ia, vb, ib):
    pred = (va > vb) | ((va == vb) & (ia > ib))
    return (jnp.where(pred, vb, va), jnp.where(pred, ib, ia),
            jnp.where(pred, va, vb), jnp.where(pred, ia, ib))


def _topk_kernel(q_ref, p_ref, pT_ref, feat_ref, Wl_ref, bl_ref,
                 idx_ref, payload_ref):
    b = pl.program_id(0)
    q = q_ref[0]
    p_full = p_ref[0]
    pT = pT_ref[0]
    feat = feat_ref[0]

    p0 = pT[0:1, :]
    p1 = pT[1:2, :]
    p2 = pT[2:3, :]
    valid_p = (p0 != 0.0) | (p1 != 0.0) | (p2 != 0.0)
    valid_col = jnp.any(p_full != 0.0, axis=1, keepdims=True)
    valid_q = jnp.any(q != 0.0, axis=1, keepdims=True)

    d0 = q[:, 0:1] - p0
    d1 = q[:, 1:2] - p1
    d2 = q[:, 2:3] - p2
    pd = d0 * d0 + d1 * d1 + d2 * d2
    pd = jnp.where(valid_q & valid_p, pd, BIG)

    @pl.when(pl.program_id(1) == 0)
    def _build_payload():
        lifted = jax.nn.relu(jnp.dot(feat, Wl_ref[...],
                                     preferred_element_type=jnp.float32)
                             + bl_ref[...])
        lifted = jnp.where(valid_col, lifted, 0.0)
        payload_ref[0] = jnp.concatenate(
            [p_full, valid_col.astype(jnp.float32), lifted,
             jnp.zeros((N, PAD_D - 4 - C_LIFT), jnp.float32)], axis=1)

    base = jax.lax.broadcasted_iota(jnp.int32, (TILE, Q), 1).astype(jnp.float32)
    v0, i0 = pd[:, 0:Q], base
    v1, i1 = pd[:, Q:2 * Q], base + float(Q)
    v2, i2 = pd[:, 2 * Q:3 * Q], base + float(2 * Q)
    v3, i3 = pd[:, 3 * Q:4 * Q], base + float(3 * Q)
    v0, i0, v1, i1 = _ce(v0, i0, v1, i1)
    v2, i2, v3, i3 = _ce(v2, i2, v3, i3)
    v0, i0, v2, i2 = _ce(v0, i0, v2, i2)
    v1, i1, v3, i3 = _ce(v1, i1, v3, i3)
    v1, i1, v2, i2 = _ce(v1, i1, v2, i2)

    cols = []
    for _ in range(K):
        m = jnp.min(v0, axis=1, keepdims=True)
        cand = jnp.where(v0 <= m, i0, IDX_BIG)
        argi = jnp.min(cand, axis=1, keepdims=True)
        pool_oh = i0 == argi
        v0 = jnp.where(pool_oh, v1, v0)
        i0 = jnp.where(pool_oh, i1, i0)
        v1 = jnp.where(pool_oh, v2, v1)
        i1 = jnp.where(pool_oh, i2, i1)
        v2 = jnp.where(pool_oh, v3, v2)
        i2 = jnp.where(pool_oh, i3, i2)
        v3 = jnp.where(pool_oh, HUGE, v3)
        i3 = jnp.where(pool_oh, IDX_BIG, i3)
        cols.append(argi)
    idx_f = jnp.concatenate(cols, axis=1)
    idx_ref[0] = idx_f.astype(jnp.int32) + b * N


def _sc_gather(table_ref, idx_ref, out_ref, idx_v, rows_v, sem):
    wid = lax.axis_index("s") * 2 + lax.axis_index("c")
    base = wid * PER_W
    for ci in range(NCHUNK):
        start = base + ci * CHUNK
        pltpu.sync_copy(idx_ref.at[pl.ds(start, CHUNK)], idx_v)
        pltpu.async_copy(table_ref.at[idx_v], rows_v, sem).wait()
        pltpu.sync_copy(rows_v, out_ref.at[pl.ds(start, CHUNK)])


def _dense_kernel(q_ref, g_ref, W1_ref, b1_ref, W2_ref, b2_ref,
                  Wf_ref, bf_ref, out_ref):
    q = q_ref[0]
    gb = g_ref[0]
    valid_q = jnp.any(q != 0.0, axis=1, keepdims=True)

    h_parts = []
    L_parts = []
    for k in range(K):
        g = gb[k]
        nbr_valid = g[:, 3:4] > 0.5
        rel = jnp.where(nbr_valid, g[:, 0:3] - q, 0.0)
        h = jax.nn.relu(jnp.dot(rel, W1_ref[...],
                                preferred_element_type=jnp.float32)
                        + b1_ref[...])
        h_parts.append(h)
        L_parts.append(g[:, 4:4 + C_LIFT])

    h_flat = jnp.concatenate(h_parts, axis=1)
    Xp = jnp.dot(h_flat, W2_ref[...],
                 preferred_element_type=jnp.float32) + b2_ref[...]

    lane_e = jax.lax.broadcasted_iota(jnp.int32, (K, N), 1)
    row_e = jax.lax.broadcasted_iota(jnp.int32, (K, N), 0)
    E = (lane_e // C_LIFT == row_e).astype(jnp.float32)

    T = None
    for j in range(K):
        Xrep = jnp.dot(Xp[:, j * K:(j + 1) * K], E,
                       preferred_element_type=jnp.float32, precision=HI)
        Ltile = pltpu.repeat(L_parts[j], K, axis=1)
        T = Xrep * Ltile if T is None else T + Xrep * Ltile

    final = jax.nn.relu(jnp.dot(T, Wf_ref[...],
                                preferred_element_type=jnp.float32)
                        + bf_ref[...])
    out_ref[0] = jnp.where(valid_q, final, 0.0)


def _half_pipeline(points_xyz, features, W1, b1, W2p, b2p, Wl, bl, Wf, bf):
    pT = jnp.transpose(points_xyz, (0, 2, 1))

    idx, payload = pl.pallas_call(
        _topk_kernel,
        grid=(B_H, N // TILE),
        in_specs=[
            pl.BlockSpec((1, TILE, 3), lambda b, t: (b, t, 0)),
            pl.BlockSpec((1, N, 3), lambda b, t: (b, 0, 0)),
            pl.BlockSpec((1, 3, N), lambda b, t: (b, 0, 0)),
            pl.BlockSpec((1, N, C_IN), lambda b, t: (b, 0, 0)),
            pl.BlockSpec(Wl.shape, lambda b, t: (0, 0)),
            pl.BlockSpec((1, C_LIFT), lambda b, t: (0, 0)),
        ],
        out_specs=[
            pl.BlockSpec((1, TILE, K), lambda b, t: (b, t, 0)),
            pl.BlockSpec((1, N, PAD_D), lambda b, t: (b, 0, 0)),
        ],
        out_shape=[
            jax.ShapeDtypeStruct((B_H, N, K), jnp.int32),
            jax.ShapeDtypeStruct((B_H, N, PAD_D), jnp.float32),
        ],
    )(points_xyz, points_xyz, pT, features, Wl, bl.reshape(1, -1))

    table = payload.reshape(ROWS, PAD_D)
    idx_flat = jnp.transpose(idx, (0, 2, 1)).reshape(LOOKUPS)

    mesh = plsc.VectorSubcoreMesh(core_axis_name="c", subcore_axis_name="s")
    gathered = pl.kernel(
        _sc_gather,
        mesh=mesh,
        out_type=jax.ShapeDtypeStruct((LOOKUPS, PAD_D), jnp.float32),
        scratch_types=[
            pltpu.VMEM((CHUNK,), jnp.int32),
            pltpu.VMEM((CHUNK, PAD_D), jnp.float32),
            pltpu.SemaphoreType.DMA,
        ],
    )(table, idx_flat)

    g4 = gathered.reshape(B_H, K, N, PAD_D)

    out = pl.pallas_call(
        _dense_kernel,
        grid=(B_H, N // TILE),
        in_specs=[
            pl.BlockSpec((1, TILE, 3), lambda b, t: (b, t, 0)),
            pl.BlockSpec((1, K, TILE, PAD_D), lambda b, t: (b, 0, t, 0)),
            pl.BlockSpec(W1.shape, lambda b, t: (0, 0)),
            pl.BlockSpec((1, K * 2), lambda b, t: (0, 0)),
            pl.BlockSpec(W2p.shape, lambda b, t: (0, 0)),
            pl.BlockSpec((1, K * K), lambda b, t: (0, 0)),
            pl.BlockSpec(Wf.shape, lambda b, t: (0, 0)),
            pl.BlockSpec((1, C_OUT), lambda b, t: (0, 0)),
        ],
        out_specs=pl.BlockSpec((1, TILE, C_OUT), lambda b, t: (b, t, 0)),
        out_shape=jax.ShapeDtypeStruct((B_H, N, C_OUT), jnp.float32),
    )(points_xyz, g4, W1, b1.reshape(1, -1), W2p, b2p.reshape(1, -1),
      Wf, bf.reshape(1, -1))
    return out


@jax.jit
def kernel(points_xyz, features, W1, b1, W2, b2, Wl, bl, Wf, bf):
    perm = np.arange(K * K).reshape(K, K).T.reshape(-1)
    W2p = W2[:, perm]
    b2p = b2[perm]
    halves = [
        _half_pipeline(points_xyz[h * B_H:(h + 1) * B_H],
                       features[h * B_H:(h + 1) * B_H],
                       W1, b1, W2p, b2p, Wl, bl, Wf, bf)
        for h in range(B // B_H)
    ]
    return jnp.concatenate(halves, axis=0)


# speedup vs baseline: 2.0810x; 2.0723x over previous
import jax
import jax.numpy as jnp
import numpy as np
from jax import lax
from jax.experimental import pallas as pl
from jax.experimental.pallas import tpu as pltpu
from jax.experimental.pallas import tpu_sc as plsc

B = 8
B_H = 4
N = 1024
K = 16
C_IN = 64
C_OUT = 128
C_LIFT = 64
TILE = 512
Q = N // 4
BIG = 1e9
HUGE = 3e9
IDX_BIG = 1e9
HI = jax.lax.Precision.HIGHEST
PAD_D = 128
ROWS = B_H * N
LOOKUPS = B_H * N * K
NW = 32
PER_W = LOOKUPS // NW
CHUNK = 512
NCHUNK = PER_W // CHUNK


def _ce(va, ia, vb, ib):
    pred = (va > vb) | ((va == vb) & (ia > ib))
    return (jnp.where(pred, vb, va), jnp.where(pred, ib, ia),
            jnp.where(pred, va, vb), jnp.where(pred, ia, ib))


def _topk_kernel(q_ref, p_ref, pT_ref, feat_ref, Wl_ref, bl_ref,
                 idx_ref, payload_ref):
    b = pl.program_id(0)
    q = q_ref[0]
    p_full = p_ref[0]
    pT = pT_ref[0]
    feat = feat_ref[0]

    p0 = pT[0:1, :]
    p1 = pT[1:2, :]
    p2 = pT[2:3, :]
    valid_p = (p0 != 0.0) | (p1 != 0.0) | (p2 != 0.0)
    valid_col = jnp.any(p_full != 0.0, axis=1, keepdims=True)
    valid_q = jnp.any(q != 0.0, axis=1, keepdims=True)

    d0 = q[:, 0:1] - p0
    d1 = q[:, 1:2] - p1
    d2 = q[:, 2:3] - p2
    pd = d0 * d0 + d1 * d1 + d2 * d2
    pd = jnp.where(valid_q & valid_p, pd, BIG)

    @pl.when(pl.program_id(1) == 0)
    def _build_payload():
        lifted = jax.nn.relu(jnp.dot(feat, Wl_ref[...],
                                     preferred_element_type=jnp.float32)
                             + bl_ref[...])
        lifted = jnp.where(valid_col, lifted, 0.0)
        payload_ref[0] = jnp.concatenate(
            [p_full, valid_col.astype(jnp.float32), lifted,
             jnp.zeros((N, PAD_D - 4 - C_LIFT), jnp.float32)], axis=1)

    base = jax.lax.broadcasted_iota(jnp.int32, (TILE, Q), 1).astype(jnp.float32)
    v0, i0 = pd[:, 0:Q], base
    v1, i1 = pd[:, Q:2 * Q], base + float(Q)
    v2, i2 = pd[:, 2 * Q:3 * Q], base + float(2 * Q)
    v3, i3 = pd[:, 3 * Q:4 * Q], base + float(3 * Q)
    v0, i0, v1, i1 = _ce(v0, i0, v1, i1)
    v2, i2, v3, i3 = _ce(v2, i2, v3, i3)
    v0, i0, v2, i2 = _ce(v0, i0, v2, i2)
    v1, i1, v3, i3 = _ce(v1, i1, v3, i3)
    v1, i1, v2, i2 = _ce(v1, i1, v2, i2)

    cols = []
    for _ in range(K):
        m = jnp.min(v0, axis=1, keepdims=True)
        cand = jnp.where(v0 <= m, i0, IDX_BIG)
        argi = jnp.min(cand, axis=1, keepdims=True)
        pool_oh = i0 == argi
        v0 = jnp.where(pool_oh, v1, v0)
        i0 = jnp.where(pool_oh, i1, i0)
        v1 = jnp.where(pool_oh, v2, v1)
        i1 = jnp.where(pool_oh, i2, i1)
        v2 = jnp.where(pool_oh, v3, v2)
        i2 = jnp.where(pool_oh, i3, i2)
        v3 = jnp.where(pool_oh, HUGE, v3)
        i3 = jnp.where(pool_oh, IDX_BIG, i3)
        cols.append(argi)
    idx_f = jnp.concatenate(cols, axis=1)
    idx_ref[0] = idx_f.astype(jnp.int32) + b * N


def _sc_gather(table_ref, idx_ref, out_ref, idx_v, rows_v, sem):
    wid = lax.axis_index("s") * 2 + lax.axis_index("c")
    base = wid * PER_W
    for ci in range(NCHUNK):
        start = base + ci * CHUNK
        pltpu.sync_copy(idx_ref.at[pl.ds(start, CHUNK)], idx_v)
        pltpu.async_copy(table_ref.at[idx_v], rows_v, sem).wait()
        pltpu.sync_copy(rows_v, out_ref.at[pl.ds(start, CHUNK)])


def _dense_kernel(q_ref, g_ref, W1_ref, b1_ref, W2_ref, b2_ref,
                  Wf_ref, bf_ref, out_ref):
    q = q_ref[0]
    gb = g_ref[0]
    valid_q = jnp.any(q != 0.0, axis=1, keepdims=True)

    h_parts = []
    L_parts = []
    for k in range(K):
        g = gb[k]
        nbr_valid = g[:, 3:4] > 0.5
        rel = jnp.where(nbr_valid, g[:, 0:3] - q, 0.0)
        h = jax.nn.relu(jnp.dot(rel, W1_ref[...],
                                preferred_element_type=jnp.float32)
                        + b1_ref[...])
        h_parts.append(h)
        L_parts.append(g[:, 4:4 + C_LIFT])

    h_flat = jnp.concatenate(h_parts, axis=1)
    Xp = jnp.dot(h_flat, W2_ref[...],
                 preferred_element_type=jnp.float32) + b2_ref[...]

    lane_e = jax.lax.broadcasted_iota(jnp.int32, (K, N), 1)
    row_e = jax.lax.broadcasted_iota(jnp.int32, (K, N), 0)
    E = (lane_e // C_LIFT == row_e).astype(jnp.float32)

    T = None
    for j in range(K):
        Xrep = jnp.dot(Xp[:, j * K:(j + 1) * K], E,
                       preferred_element_type=jnp.float32)
        Ltile = pltpu.repeat(L_parts[j], K, axis=1)
        T = Xrep * Ltile if T is None else T + Xrep * Ltile

    final = jax.nn.relu(jnp.dot(T, Wf_ref[...],
                                preferred_element_type=jnp.float32)
                        + bf_ref[...])
    out_ref[0] = jnp.where(valid_q, final, 0.0)


def _half_pipeline(points_xyz, features, W1, b1, W2p, b2p, Wl, bl, Wf, bf):
    pT = jnp.transpose(points_xyz, (0, 2, 1))

    idx, payload = pl.pallas_call(
        _topk_kernel,
        grid=(B_H, N // TILE),
        in_specs=[
            pl.BlockSpec((1, TILE, 3), lambda b, t: (b, t, 0)),
            pl.BlockSpec((1, N, 3), lambda b, t: (b, 0, 0)),
            pl.BlockSpec((1, 3, N), lambda b, t: (b, 0, 0)),
            pl.BlockSpec((1, N, C_IN), lambda b, t: (b, 0, 0)),
            pl.BlockSpec(Wl.shape, lambda b, t: (0, 0)),
            pl.BlockSpec((1, C_LIFT), lambda b, t: (0, 0)),
        ],
        out_specs=[
            pl.BlockSpec((1, TILE, K), lambda b, t: (b, t, 0)),
            pl.BlockSpec((1, N, PAD_D), lambda b, t: (b, 0, 0)),
        ],
        out_shape=[
            jax.ShapeDtypeStruct((B_H, N, K), jnp.int32),
            jax.ShapeDtypeStruct((B_H, N, PAD_D), jnp.float32),
        ],
    )(points_xyz, points_xyz, pT, features, Wl, bl.reshape(1, -1))

    table = payload.reshape(ROWS, PAD_D)
    idx_flat = jnp.transpose(idx, (0, 2, 1)).reshape(LOOKUPS)

    mesh = plsc.VectorSubcoreMesh(core_axis_name="c", subcore_axis_name="s")
    gathered = pl.kernel(
        _sc_gather,
        mesh=mesh,
        out_type=jax.ShapeDtypeStruct((LOOKUPS, PAD_D), jnp.float32),
        scratch_types=[
            pltpu.VMEM((CHUNK,), jnp.int32),
            pltpu.VMEM((CHUNK, PAD_D), jnp.float32),
            pltpu.SemaphoreType.DMA,
        ],
    )(table, idx_flat)

    g4 = gathered.reshape(B_H, K, N, PAD_D)

    out = pl.pallas_call(
        _dense_kernel,
        grid=(B_H, N // TILE),
        in_specs=[
            pl.BlockSpec((1, TILE, 3), lambda b, t: (b, t, 0)),
            pl.BlockSpec((1, K, TILE, PAD_D), lambda b, t: (b, 0, t, 0)),
            pl.BlockSpec(W1.shape, lambda b, t: (0, 0)),
            pl.BlockSpec((1, K * 2), lambda b, t: (0, 0)),
            pl.BlockSpec(W2p.shape, lambda b, t: (0, 0)),
            pl.BlockSpec((1, K * K), lambda b, t: (0, 0)),
            pl.BlockSpec(Wf.shape, lambda b, t: (0, 0)),
            pl.BlockSpec((1, C_OUT), lambda b, t: (0, 0)),
        ],
        out_specs=pl.BlockSpec((1, TILE, C_OUT), lambda b, t: (b, t, 0)),
        out_shape=jax.ShapeDtypeStruct((B_H, N, C_OUT), jnp.float32),
    )(points_xyz, g4, W1, b1.reshape(1, -1), W2p, b2p.reshape(1, -1),
      Wf, bf.reshape(1, -1))
    return out


@jax.jit
def kernel(points_xyz, features, W1, b1, W2, b2, Wl, bl, Wf, bf):
    perm = np.arange(K * K).reshape(K, K).T.reshape(-1)
    W2p = W2[:, perm]
    b2p = b2[perm]
    halves = [
        _half_pipeline(points_xyz[h * B_H:(h + 1) * B_H],
                       features[h * B_H:(h + 1) * B_H],
                       W1, b1, W2p, b2p, Wl, bl, Wf, bf)
        for h in range(B // B_H)
    ]
    return jnp.concatenate(halves, axis=0)


# speedup vs baseline: 2.0836x; 1.0012x over previous
import jax
import jax.numpy as jnp
import numpy as np
from jax import lax
from jax.experimental import pallas as pl
from jax.experimental.pallas import tpu as pltpu
from jax.experimental.pallas import tpu_sc as plsc

B = 8
B_H = 4
N = 1024
K = 16
C_IN = 64
C_OUT = 128
C_LIFT = 64
TILE = 512
Q = N // 4
BIG = 1e9
HUGE = 3e9
IDX_BIG = 1e9
PAD_D = 128
ROWS = B_H * N
LOOKUPS = B_H * N * K
NW = 32
PER_W = LOOKUPS // NW
CHUNK = 512
NCHUNK = PER_W // CHUNK


def _ce(va, ia, vb, ib):
    pred = (va > vb) | ((va == vb) & (ia > ib))
    return (jnp.where(pred, vb, va), jnp.where(pred, ib, ia),
            jnp.where(pred, va, vb), jnp.where(pred, ia, ib))


def _topk_kernel(q_ref, p_ref, pT_ref, feat_ref, Wl_ref, bl_ref,
                 idx_ref, payload_ref):
    b = pl.program_id(0)
    q = q_ref[0]
    p_full = p_ref[0]
    pT = pT_ref[0]
    feat = feat_ref[0]

    p0 = pT[0:1, :]
    p1 = pT[1:2, :]
    p2 = pT[2:3, :]
    valid_p = (p0 != 0.0) | (p1 != 0.0) | (p2 != 0.0)
    valid_col = jnp.any(p_full != 0.0, axis=1, keepdims=True)
    valid_q = jnp.any(q != 0.0, axis=1, keepdims=True)

    d0 = q[:, 0:1] - p0
    d1 = q[:, 1:2] - p1
    d2 = q[:, 2:3] - p2
    pd = d0 * d0 + d1 * d1 + d2 * d2
    pd = jnp.where(valid_q & valid_p, pd, BIG)

    @pl.when(pl.program_id(1) == 0)
    def _build_payload():
        lifted = jax.nn.relu(jnp.dot(feat, Wl_ref[...],
                                     preferred_element_type=jnp.float32)
                             + bl_ref[...])
        lifted = jnp.where(valid_col, lifted, 0.0)
        payload_ref[0] = jnp.concatenate(
            [p_full, valid_col.astype(jnp.float32), lifted,
             jnp.zeros((N, PAD_D - 4 - C_LIFT), jnp.float32)], axis=1)

    base = jax.lax.broadcasted_iota(jnp.int32, (TILE, Q), 1).astype(jnp.float32)
    v0, i0 = pd[:, 0:Q], base
    v1, i1 = pd[:, Q:2 * Q], base + float(Q)
    v2, i2 = pd[:, 2 * Q:3 * Q], base + float(2 * Q)
    v3, i3 = pd[:, 3 * Q:4 * Q], base + float(3 * Q)
    v0, i0, v1, i1 = _ce(v0, i0, v1, i1)
    v2, i2, v3, i3 = _ce(v2, i2, v3, i3)
    v0, i0, v2, i2 = _ce(v0, i0, v2, i2)
    v1, i1, v3, i3 = _ce(v1, i1, v3, i3)
    v1, i1, v2, i2 = _ce(v1, i1, v2, i2)

    cols = []
    for _ in range(K):
        m = jnp.min(v0, axis=1, keepdims=True)
        cand = jnp.where(v0 <= m, i0, IDX_BIG)
        argi = jnp.min(cand, axis=1, keepdims=True)
        pool_oh = i0 == argi
        v0 = jnp.where(pool_oh, v1, v0)
        i0 = jnp.where(pool_oh, i1, i0)
        v1 = jnp.where(pool_oh, v2, v1)
        i1 = jnp.where(pool_oh, i2, i1)
        v2 = jnp.where(pool_oh, v3, v2)
        i2 = jnp.where(pool_oh, i3, i2)
        v3 = jnp.where(pool_oh, HUGE, v3)
        i3 = jnp.where(pool_oh, IDX_BIG, i3)
        cols.append(argi)
    idx_f = jnp.concatenate(cols, axis=1)
    idx_ref[0] = idx_f.astype(jnp.int32) + b * N


def _sc_gather(table_ref, idx_ref, out_ref, idx_v, rows_v, sem):
    wid = lax.axis_index("s") * 2 + lax.axis_index("c")
    base = wid * PER_W
    for ci in range(NCHUNK):
        start = base + ci * CHUNK
        pltpu.sync_copy(idx_ref.at[pl.ds(start, CHUNK)], idx_v)
        pltpu.async_copy(table_ref.at[idx_v], rows_v, sem).wait()
        pltpu.sync_copy(rows_v, out_ref.at[pl.ds(start, CHUNK)])


def _dense_kernel(q_ref, g_ref, W1_ref, b1_ref, W2_ref, b2_ref,
                  Wf_ref, bf_ref, out_ref):
    q = q_ref[0]
    gb = g_ref[0]
    valid_q = jnp.any(q != 0.0, axis=1, keepdims=True)

    h_parts = []
    L_parts = []
    for k in range(K):
        g = gb[k]
        nbr_valid = g[:, 3:4] > 0.5
        rel = jnp.where(nbr_valid, g[:, 0:3] - q, 0.0)
        h = jax.nn.relu(jnp.dot(rel, W1_ref[...],
                                preferred_element_type=jnp.float32)
                        + b1_ref[...])
        h_parts.append(h)
        L_parts.append(g[:, 4:4 + C_LIFT])

    h_flat = jnp.concatenate(h_parts, axis=1)
    Xp = jnp.dot(h_flat, W2_ref[...],
                 preferred_element_type=jnp.float32) + b2_ref[...]

    lane_e = jax.lax.broadcasted_iota(jnp.int32, (K, N), 1)
    row_e = jax.lax.broadcasted_iota(jnp.int32, (K, N), 0)
    E = (lane_e // C_LIFT == row_e).astype(jnp.float32)

    T = None
    for j in range(K):
        Xrep = jnp.dot(Xp[:, j * K:(j + 1) * K], E,
                       preferred_element_type=jnp.float32)
        Ltile = pltpu.repeat(L_parts[j], K, axis=1)
        T = Xrep * Ltile if T is None else T + Xrep * Ltile

    final = jax.nn.relu(jnp.dot(T, Wf_ref[...],
                                preferred_element_type=jnp.float32)
                        + bf_ref[...])
    out_ref[0] = jnp.where(valid_q, final, 0.0)


def _half_pipeline(points_xyz, features, W1, b1, W2p, b2p, Wl, bl, Wf, bf):
    pT = jnp.transpose(points_xyz, (0, 2, 1))

    idx, payload = pl.pallas_call(
        _topk_kernel,
        grid=(B_H, N // TILE),
        in_specs=[
            pl.BlockSpec((1, TILE, 3), lambda b, t: (b, t, 0)),
            pl.BlockSpec((1, N, 3), lambda b, t: (b, 0, 0)),
            pl.BlockSpec((1, 3, N), lambda b, t: (b, 0, 0)),
            pl.BlockSpec((1, N, C_IN), lambda b, t: (b, 0, 0)),
            pl.BlockSpec(Wl.shape, lambda b, t: (0, 0)),
            pl.BlockSpec((1, C_LIFT), lambda b, t: (0, 0)),
        ],
        out_specs=[
            pl.BlockSpec((1, TILE, K), lambda b, t: (b, t, 0)),
            pl.BlockSpec((1, N, PAD_D), lambda b, t: (b, 0, 0)),
        ],
        out_shape=[
            jax.ShapeDtypeStruct((B_H, N, K), jnp.int32),
            jax.ShapeDtypeStruct((B_H, N, PAD_D), jnp.float32),
        ],
    )(points_xyz, points_xyz, pT, features, Wl, bl.reshape(1, -1))

    table = payload.reshape(ROWS, PAD_D)
    idx_flat = jnp.transpose(idx, (0, 2, 1)).reshape(LOOKUPS)

    mesh = plsc.VectorSubcoreMesh(core_axis_name="c", subcore_axis_name="s")
    gathered = pl.kernel(
        _sc_gather,
        mesh=mesh,
        out_type=jax.ShapeDtypeStruct((LOOKUPS, PAD_D), jnp.float32),
        scratch_types=[
            pltpu.VMEM((CHUNK,), jnp.int32),
            pltpu.VMEM((CHUNK, PAD_D), jnp.float32),
            pltpu.SemaphoreType.DMA,
        ],
    )(table, idx_flat)

    g4 = gathered.reshape(B_H, K, N, PAD_D)

    out = pl.pallas_call(
        _dense_kernel,
        grid=(B_H, N // TILE),
        in_specs=[
            pl.BlockSpec((1, TILE, 3), lambda b, t: (b, t, 0)),
            pl.BlockSpec((1, K, TILE, PAD_D), lambda b, t: (b, 0, t, 0)),
            pl.BlockSpec(W1.shape, lambda b, t: (0, 0)),
            pl.BlockSpec((1, K * 2), lambda b, t: (0, 0)),
            pl.BlockSpec(W2p.shape, lambda b, t: (0, 0)),
            pl.BlockSpec((1, K * K), lambda b, t: (0, 0)),
            pl.BlockSpec(Wf.shape, lambda b, t: (0, 0)),
            pl.BlockSpec((1, C_OUT), lambda b, t: (0, 0)),
        ],
        out_specs=pl.BlockSpec((1, TILE, C_OUT), lambda b, t: (b, t, 0)),
        out_shape=jax.ShapeDtypeStruct((B_H, N, C_OUT), jnp.float32),
    )(points_xyz, g4, W1, b1.reshape(1, -1), W2p, b2p.reshape(1, -1),
      Wf, bf.reshape(1, -1))
    return out


@jax.jit
def kernel(points_xyz, features, W1, b1, W2, b2, Wl, bl, Wf, bf):
    perm = np.arange(K * K).reshape(K, K).T.reshape(-1)
    W2p = W2[:, perm]
    b2p = b2[perm]
    halves = [
        _half_pipeline(points_xyz[h * B_H:(h + 1) * B_H],
                       features[h * B_H:(h + 1) * B_H],
                       W1, b1, W2p, b2p, Wl, bl, Wf, bf)
        for h in range(B // B_H)
    ]
    return jnp.concatenate(halves, axis=0)
